# baseline (speedup 1.0000x reference)
import functools
import math

import jax
import jax.numpy as jnp
import numpy as np
from jax.experimental import pallas as pl
from jax.experimental.pallas import tpu as pltpu

_FACTOR = 5


@functools.cache
def _sample_constants_host(L, S, u_part):
    with jax.ensure_compile_time_eval():
        idx = np.asarray(jax.random.randint(
            jax.random.key(42), (L, u_part), 0, S))
    cnt = np.zeros((S, L), np.float32)
    np.add.at(cnt, (idx.ravel(), np.repeat(np.arange(L), u_part)), 1.0)
    bias = np.where(cnt > 0, np.float32(0), np.float32(-np.inf))
    return bias, cnt.astype(np.float32)


def _sample_constants(L, S, u_part):
    try:
        bias, cnt = _sample_constants_host(L, S, u_part)
        return jnp.asarray(bias), jnp.asarray(cnt)
    except Exception:
        idx = jax.random.randint(jax.random.key(42), (L, u_part), 0, S)
        cnt = jnp.zeros((S, L), jnp.float32).at[
            idx.T, jnp.arange(L)[None, :]].add(1.0)
        bias = jnp.where(cnt > 0, jnp.float32(0), jnp.float32(-jnp.inf))
        return bias, cnt


def _t_kernel(q_ref, k_ref, v_ref, qo_ref, ko_ref, vo_ref):
    qo_ref[...] = jnp.swapaxes(q_ref[...], 0, 1)
    ko_ref[...] = jnp.swapaxes(k_ref[...], 0, 1)
    vo_ref[...] = jnp.swapaxes(v_ref[...], 0, 1)


def _m_kernel(bias_ref, cnt_ref, q_ref, k_ref, m_ref, *, L, S, TS):
    q = q_ref[...]
    neg = jnp.float32(-jnp.inf)
    run_max = jnp.full((1, L), neg, dtype=jnp.float32)
    run_sum = jnp.zeros((1, L), dtype=jnp.float32)
    for t in range(S // TS):
        k_t = k_ref[t * TS:(t + 1) * TS, :]
        s_t = jax.lax.dot_general(k_t, q, (((1,), (1,)), ((), ())),
                                  preferred_element_type=jnp.float32)
        masked = s_t + bias_ref[t * TS:(t + 1) * TS, :]
        run_max = jnp.maximum(run_max, jnp.max(masked, axis=0, keepdims=True))
        run_sum = run_sum + jnp.sum(
            s_t * cnt_ref[t * TS:(t + 1) * TS, :], axis=0, keepdims=True)
    m_ref[0, :] = (run_max - run_sum * (1.0 / S))[0, :]


def _topk_kernel(m_ref, idx_ref, *, BH, L, n_top):
    m = m_ref[:, 0, :]
    neg = jnp.float32(-jnp.inf)
    iota = jax.lax.broadcasted_iota(jnp.int32, (BH, L), 1)
    rank = jax.lax.broadcasted_iota(jnp.int32, (BH, n_top), 1)
    idxes = jnp.zeros((BH, n_top), jnp.int32)
    for i in range(n_top):
        mv = jnp.max(m, axis=1, keepdims=True)
        fidx = jnp.min(jnp.where(m == mv, iota, L), axis=1,
                       keepdims=True)
        idxes = jnp.where(rank == i, fidx, idxes)
        m = jnp.where(iota == fidx, neg, m)
    idx_ref[...] = idxes


def _attn_kernel(idx_ref, q_ref, k_ref, v_ref, ctx_ref, w_ref, qr_ref,
                 *, n_top, scale):
    i = pl.program_id(0)
    for r in range(n_top):
        qr_ref[pl.ds(r, 1), :] = q_ref[pl.ds(idx_ref[i, r], 1), :]
    qr = qr_ref[...]
    sc = jax.lax.dot_general(qr, k_ref[...], (((1,), (1,)), ((), ())),
                             preferred_element_type=jnp.float32) * scale
    mx = jnp.max(sc, axis=1, keepdims=True)
    e = jnp.exp(sc - mx)
    w = e / jnp.sum(e, axis=1, keepdims=True)
    w_ref[...] = w
    ctx_ref[...] = jnp.dot(w, v_ref[...], preferred_element_type=jnp.float32)


def kernel(queries, keys, values):
    B, L, H, D = queries.shape
    S = keys.shape[1]

    U = _FACTOR * int(np.ceil(np.log(S)))
    u = _FACTOR * int(np.ceil(np.log(L)))
    n_top = min(U, L)
    u_part = min(u, S)

    bias, cnt = _sample_constants(L, S, u_part)

    BH = B * H
    TL = 256
    Q4, K4, V4 = pl.pallas_call(
        _t_kernel,
        grid=(B, L // TL),
        in_specs=[pl.BlockSpec((None, TL, H, D), lambda b, t: (b, t, 0, 0))] * 3,
        out_specs=[pl.BlockSpec((None, H, TL, D), lambda b, t: (b, 0, t, 0))] * 3,
        out_shape=[jax.ShapeDtypeStruct((B, H, L, D), jnp.float32)] * 3,
        compiler_params=pltpu.CompilerParams(
            dimension_semantics=("arbitrary", "arbitrary"),
        ),
    )(queries, keys, values)
    Q = Q4.reshape(BH, L, D)
    K = K4.reshape(BH, S, D)
    V = V4.reshape(BH, S, D)

    scale = 1.0 / math.sqrt(D)
    TS = 512 if S % 512 == 0 else S

    m = pl.pallas_call(
        functools.partial(_m_kernel, L=L, S=S, TS=TS),
        grid=(BH,),
        in_specs=[
            pl.BlockSpec((S, L), lambda i: (0, 0)),
            pl.BlockSpec((S, L), lambda i: (0, 0)),
            pl.BlockSpec((None, L, D), lambda i: (i, 0, 0)),
            pl.BlockSpec((None, S, D), lambda i: (i, 0, 0)),
        ],
        out_specs=pl.BlockSpec((None, 1, L), lambda i: (i, 0, 0)),
        out_shape=jax.ShapeDtypeStruct((BH, 1, L), jnp.float32),
        compiler_params=pltpu.CompilerParams(
            dimension_semantics=("arbitrary",),
        ),
    )(bias, cnt, Q, K)

    idx = pl.pallas_call(
        functools.partial(_topk_kernel, BH=BH, L=L, n_top=n_top),
        in_specs=[pl.BlockSpec((BH, 1, L), lambda: (0, 0, 0))],
        out_specs=pl.BlockSpec((BH, n_top), lambda: (0, 0)),
        out_shape=jax.ShapeDtypeStruct((BH, n_top), jnp.int32),
    )(m)

    ctx, w = pl.pallas_call(
        functools.partial(_attn_kernel, n_top=n_top, scale=scale),
        grid=(BH,),
        in_specs=[
            pl.BlockSpec(memory_space=pltpu.SMEM),
            pl.BlockSpec((None, L, D), lambda i: (i, 0, 0)),
            pl.BlockSpec((None, S, D), lambda i: (i, 0, 0)),
            pl.BlockSpec((None, S, D), lambda i: (i, 0, 0)),
        ],
        out_specs=[
            pl.BlockSpec((None, n_top, D), lambda i: (i, 0, 0)),
            pl.BlockSpec((None, n_top, S), lambda i: (i, 0, 0)),
        ],
        out_shape=[
            jax.ShapeDtypeStruct((BH, n_top, D), jnp.float32),
            jax.ShapeDtypeStruct((BH, n_top, S), jnp.float32),
        ],
        scratch_shapes=[pltpu.VMEM((n_top, D), jnp.float32)],
        compiler_params=pltpu.CompilerParams(
            dimension_semantics=("arbitrary",),
        ),
    )(idx, Q, K, V)

    return (ctx.reshape(B, H, n_top, D), w.reshape(B, H, n_top, S))


# speedup vs baseline: 1.4446x; 1.4446x over previous
import functools
import math

import jax
import jax.numpy as jnp
import numpy as np
from jax.experimental import pallas as pl
from jax.experimental.pallas import tpu as pltpu

_FACTOR = 5


@functools.cache
def _sample_constants_host(L, S, u_part):
    with jax.ensure_compile_time_eval():
        idx = np.asarray(jax.random.randint(
            jax.random.key(42), (L, u_part), 0, S))
    cnt = np.zeros((S, L), np.float32)
    np.add.at(cnt, (idx.ravel(), np.repeat(np.arange(L), u_part)), 1.0)
    bias = np.where(cnt > 0, np.float32(0), np.float32(-np.inf))
    return bias, cnt.astype(np.float32)


def _sample_constants(L, S, u_part):
    try:
        bias, cnt = _sample_constants_host(L, S, u_part)
        return jnp.asarray(bias), jnp.asarray(cnt)
    except Exception:
        idx = jax.random.randint(jax.random.key(42), (L, u_part), 0, S)
        cnt = jnp.zeros((S, L), jnp.float32).at[
            idx.T, jnp.arange(L)[None, :]].add(1.0)
        bias = jnp.where(cnt > 0, jnp.float32(0), jnp.float32(-jnp.inf))
        return bias, cnt


def _m_topk_kernel(bias_ref, cnt_ref, q_ref, k_ref, idx_ref, macc_ref,
                   *, BH, L, S, TS, n_top):
    i = pl.program_id(0)
    q = q_ref[...]
    neg = jnp.float32(-jnp.inf)
    run_max = jnp.full((1, L), neg, dtype=jnp.float32)
    run_sum = jnp.zeros((1, L), dtype=jnp.float32)
    for t in range(S // TS):
        k_t = k_ref[t * TS:(t + 1) * TS, :]
        s_t = jax.lax.dot_general(k_t, q, (((1,), (1,)), ((), ())),
                                  preferred_element_type=jnp.float32)
        masked = s_t + bias_ref[t * TS:(t + 1) * TS, :]
        run_max = jnp.maximum(run_max, jnp.max(masked, axis=0, keepdims=True))
        run_sum = run_sum + jnp.sum(
            s_t * cnt_ref[t * TS:(t + 1) * TS, :], axis=0, keepdims=True)
    macc_ref[pl.ds(i, 1), :] = run_max - run_sum * (1.0 / S)

    @pl.when(i == BH - 1)
    def _():
        m = macc_ref[...]
        iota = jax.lax.broadcasted_iota(jnp.int32, (BH, L), 1)
        rank = jax.lax.broadcasted_iota(jnp.int32, (BH, n_top), 1)
        idxes = jnp.zeros((BH, n_top), jnp.int32)
        for r in range(n_top):
            mv = jnp.max(m, axis=1, keepdims=True)
            fidx = jnp.min(jnp.where(m == mv, iota, L), axis=1,
                           keepdims=True)
            idxes = jnp.where(rank == r, fidx, idxes)
            m = jnp.where(iota == fidx, neg, m)
        idx_ref[...] = idxes


def _attn_kernel(idx_ref, q_ref, k_ref, v_ref, ctx_ref, w_ref, qr_ref,
                 *, n_top, scale):
    i = pl.program_id(0)
    for r in range(n_top):
        qr_ref[pl.ds(r, 1), :] = q_ref[pl.ds(idx_ref[i, r], 1), :]
    qr = qr_ref[...]
    sc = jax.lax.dot_general(qr, k_ref[...], (((1,), (1,)), ((), ())),
                             preferred_element_type=jnp.float32) * scale
    mx = jnp.max(sc, axis=1, keepdims=True)
    e = jnp.exp(sc - mx)
    w = e / jnp.sum(e, axis=1, keepdims=True)
    w_ref[...] = w
    ctx_ref[...] = jnp.dot(w, v_ref[...], preferred_element_type=jnp.float32)


def kernel(queries, keys, values):
    B, L, H, D = queries.shape
    S = keys.shape[1]

    U = _FACTOR * int(np.ceil(np.log(S)))
    u = _FACTOR * int(np.ceil(np.log(L)))
    n_top = min(U, L)
    u_part = min(u, S)

    bias, cnt = _sample_constants(L, S, u_part)

    BH = B * H
    Q = jnp.transpose(queries, (0, 2, 1, 3)).reshape(BH, L, D)
    K = jnp.transpose(keys, (0, 2, 1, 3)).reshape(BH, S, D)
    V = jnp.transpose(values, (0, 2, 1, 3)).reshape(BH, S, D)

    scale = 1.0 / math.sqrt(D)
    TS = 1024 if S % 1024 == 0 else S

    idx = pl.pallas_call(
        functools.partial(_m_topk_kernel, BH=BH, L=L, S=S, TS=TS,
                          n_top=n_top),
        grid=(BH,),
        in_specs=[
            pl.BlockSpec((S, L), lambda i: (0, 0)),
            pl.BlockSpec((S, L), lambda i: (0, 0)),
            pl.BlockSpec((None, L, D), lambda i: (i, 0, 0)),
            pl.BlockSpec((None, S, D), lambda i: (i, 0, 0)),
        ],
        out_specs=pl.BlockSpec((BH, n_top), lambda i: (0, 0)),
        out_shape=jax.ShapeDtypeStruct((BH, n_top), jnp.int32),
        scratch_shapes=[pltpu.VMEM((BH, L), jnp.float32)],
        compiler_params=pltpu.CompilerParams(
            dimension_semantics=("arbitrary",),
        ),
    )(bias, cnt, Q, K)

    ctx, w = pl.pallas_call(
        functools.partial(_attn_kernel, n_top=n_top, scale=scale),
        grid=(BH,),
        in_specs=[
            pl.BlockSpec(memory_space=pltpu.SMEM),
            pl.BlockSpec((None, L, D), lambda i: (i, 0, 0)),
            pl.BlockSpec((None, S, D), lambda i: (i, 0, 0)),
            pl.BlockSpec((None, S, D), lambda i: (i, 0, 0)),
        ],
        out_specs=[
            pl.BlockSpec((None, n_top, D), lambda i: (i, 0, 0)),
            pl.BlockSpec((None, n_top, S), lambda i: (i, 0, 0)),
        ],
        out_shape=[
            jax.ShapeDtypeStruct((BH, n_top, D), jnp.float32),
            jax.ShapeDtypeStruct((BH, n_top, S), jnp.float32),
        ],
        scratch_shapes=[pltpu.VMEM((n_top, D), jnp.float32)],
        compiler_params=pltpu.CompilerParams(
            dimension_semantics=("arbitrary",),
        ),
    )(idx, Q, K, V)

    return (ctx.reshape(B, H, n_top, D), w.reshape(B, H, n_top, S))
